# baseline (speedup 1.0000x reference)
import functools
import jax
import jax.numpy as jnp
from jax import lax
from jax.experimental import pallas as pl
from jax.experimental.pallas import tpu as pltpu
from jax.experimental.pallas import tpu_sc as plsc

D_MODEL = 64
SCALE = 8.0
IDX_MINOR = 128


def _make_sc_lookup(B):
    NC, NS = 2, 16
    NW = NC * NS
    per_w = B // NW
    CH = 512
    n_chunk = per_w // CH
    IR = CH // IDX_MINOR

    mesh = plsc.VectorSubcoreMesh(core_axis_name="c", subcore_axis_name="s")

    @functools.partial(
        pl.kernel,
        mesh=mesh,
        out_type=jax.ShapeDtypeStruct((B, D_MODEL), jnp.float32),
        scratch_types=[
            pltpu.VMEM((IR, IDX_MINOR), jnp.int32),
            pltpu.VMEM((CH, D_MODEL), jnp.float32),
            pltpu.SemaphoreType.DMA,
        ],
        compiler_params=pltpu.CompilerParams(use_tc_tiling_on_sc=False),
    )
    def lookup(table_hbm, idx_hbm, out_hbm, idx_v, rows_v, sem):
        wid = lax.axis_index("s") * NC + lax.axis_index("c")

        def chunk_body(g, carry):
            blk = wid * n_chunk + g
            pltpu.sync_copy(idx_hbm.at[pl.ds(blk * IR, IR)], idx_v)
            copies = [
                pltpu.async_copy(
                    table_hbm.at[idx_v.at[j]],
                    rows_v.at[pl.ds(j * IDX_MINOR, IDX_MINOR)],
                    sem,
                )
                for j in range(IR)
            ]
            for c in copies:
                c.wait()

            def scale_body(r, _):
                for j in range(D_MODEL // 16):
                    sl = pl.ds(j * 16, 16)
                    rows_v[r, sl] = rows_v[r, sl] * SCALE
                return 0

            lax.fori_loop(0, CH, scale_body, 0)
            pltpu.sync_copy(rows_v, out_hbm.at[pl.ds(blk * CH, CH)])
            return carry

        lax.fori_loop(0, n_chunk, chunk_body, 0)

    return lookup


def kernel(input_sequence, embedding_table):
    S, T = input_sequence.shape
    B = S * T
    idx = input_sequence.reshape(B // IDX_MINOR, IDX_MINOR).astype(jnp.int32)
    out = _make_sc_lookup(B)(embedding_table, idx)
    return out.reshape(S, T, D_MODEL)


# speedup vs baseline: 1.1374x; 1.1374x over previous
import functools
import jax
import jax.numpy as jnp
from jax import lax
from jax.experimental import pallas as pl
from jax.experimental.pallas import tpu as pltpu
from jax.experimental.pallas import tpu_sc as plsc

D_MODEL = 64
SCALE = 8.0
IDX_MINOR = 128


def _make_sc_lookup(B):
    NC, NS = 2, 16
    NW = NC * NS
    per_w = B // NW
    CH = 256
    NBUF = 4
    AHEAD = 2
    n_chunk = per_w // CH
    IR = CH // IDX_MINOR
    idx_rows = per_w // IDX_MINOR
    n_outer = n_chunk // NBUF

    mesh = plsc.VectorSubcoreMesh(core_axis_name="c", subcore_axis_name="s")

    @functools.partial(
        pl.kernel,
        mesh=mesh,
        out_type=jax.ShapeDtypeStruct((B, D_MODEL), jnp.float32),
        scratch_types=[
            pltpu.VMEM((idx_rows, IDX_MINOR), jnp.int32),
            pltpu.VMEM((NBUF * CH, D_MODEL), jnp.float32),
            pltpu.SemaphoreType.DMA((NBUF,)),
            pltpu.SemaphoreType.DMA((NBUF,)),
        ],
        compiler_params=pltpu.CompilerParams(use_tc_tiling_on_sc=False),
    )
    def lookup(table_hbm, idx_hbm, out_hbm, idx_v, rows_v, gsem, osem):
        wid = lax.axis_index("s") * NC + lax.axis_index("c")
        pltpu.sync_copy(idx_hbm.at[pl.ds(wid * idx_rows, idx_rows)], idx_v)
        row0 = wid * per_w

        def gather_parts(g, b):
            return [
                (
                    table_hbm.at[idx_v.at[g * IR + j]],
                    rows_v.at[pl.ds(b * CH + j * IDX_MINOR, IDX_MINOR)],
                    gsem.at[b],
                )
                for j in range(IR)
            ]

        def fire_gather(g, b):
            for src, dst, sem in gather_parts(g, b):
                pltpu.async_copy(src, dst, sem)

        def wait_gather(g, b):
            for src, dst, sem in gather_parts(g, b):
                pltpu.make_async_copy(src, dst, sem).wait()

        def out_parts(g, b):
            return (
                rows_v.at[pl.ds(b * CH, CH)],
                out_hbm.at[pl.ds(row0 + g * CH, CH)],
                osem.at[b],
            )

        def fire_out(g, b):
            src, dst, sem = out_parts(g, b)
            pltpu.async_copy(src, dst, sem)

        def wait_out(g, b):
            src, dst, sem = out_parts(g, b)
            pltpu.make_async_copy(src, dst, sem).wait()

        def scale(b):
            @plsc.parallel_loop(b * CH, (b + 1) * CH, unroll=8)
            def _(r):
                for j in range(D_MODEL // 16):
                    sl = pl.ds(j * 16, 16)
                    rows_v[r, sl] = rows_v[r, sl] * SCALE

        def step(g, b, do_wait_out, do_fire):
            bb = (b + AHEAD) % NBUF
            if do_wait_out:
                wait_out(g - AHEAD, bb)
            if do_fire:
                fire_gather(g + AHEAD, bb)
            wait_gather(g, b)
            scale(b)
            fire_out(g, b)

        for g in range(AHEAD):
            fire_gather(g, g % NBUF)

        for g in range(NBUF):
            step(g, g % NBUF, do_wait_out=g >= AHEAD, do_fire=True)

        def outer_body(outer, carry):
            g0 = outer * NBUF
            for b in range(NBUF):
                step(g0 + b, b, do_wait_out=True, do_fire=True)
            return carry

        lax.fori_loop(1, n_outer - 1, outer_body, 0)

        for g in range(n_chunk - NBUF, n_chunk):
            b = g % NBUF
            step(g, b, do_wait_out=g + AHEAD - NBUF < n_chunk - NBUF,
                 do_fire=g + AHEAD < n_chunk)

        for g in range(n_chunk - NBUF, n_chunk):
            wait_out(g, g % NBUF)

    return lookup


def kernel(input_sequence, embedding_table):
    S, T = input_sequence.shape
    B = S * T
    idx = input_sequence.reshape(B // IDX_MINOR, IDX_MINOR).astype(jnp.int32)
    out = _make_sc_lookup(B)(embedding_table, idx)
    return out.reshape(S, T, D_MODEL)


# speedup vs baseline: 1.2297x; 1.0811x over previous
import functools
import jax
import jax.numpy as jnp
from jax import lax
from jax.experimental import pallas as pl
from jax.experimental.pallas import tpu as pltpu
from jax.experimental.pallas import tpu_sc as plsc

D_MODEL = 64
SCALE = 8.0
BC = 2048
NROWS = 1000000
N_PAD = 489 * BC


def _tc_repack(table_t):
    def repack_kernel(a_ref, out_ref):
        out_ref[:, 0:D_MODEL] = a_ref[...].T * SCALE

    return pl.pallas_call(
        repack_kernel,
        grid=(N_PAD // BC,),
        in_specs=[pl.BlockSpec((D_MODEL, BC), lambda b: (0, b))],
        out_specs=pl.BlockSpec((BC, 128), lambda b: (b, 0)),
        out_shape=jax.ShapeDtypeStruct((N_PAD, 128), jnp.float32),
    )(table_t)


def _make_sc_gather(T_COLS, S):
    NC, NS = 2, 16
    NW = NC * NS
    SB = S // NW
    n_t = T_COLS
    GBUF = 4
    AHEAD = 2
    OBUF = 2

    mesh = plsc.VectorSubcoreMesh(core_axis_name="c", subcore_axis_name="s")

    @functools.partial(
        pl.kernel,
        mesh=mesh,
        out_type=jax.ShapeDtypeStruct((T_COLS, D_MODEL, S), jnp.float32),
        scratch_types=[
            pltpu.VMEM((GBUF, SB), jnp.int32),
            pltpu.VMEM((GBUF, SB, 128), jnp.float32),
            pltpu.VMEM((OBUF, D_MODEL, SB), jnp.float32),
            pltpu.SemaphoreType.DMA((GBUF,)),
            pltpu.SemaphoreType.DMA((GBUF,)),
            pltpu.SemaphoreType.DMA((OBUF,)),
        ],
        compiler_params=pltpu.CompilerParams(
            use_tc_tiling_on_sc=True, needs_layout_passes=False
        ),
    )
    def gather_k(table_hbm, idx_hbm, out_hbm, isr, gbuf, tbuf,
                 isem, gsem, osem):
        wid = lax.axis_index("s") * NC + lax.axis_index("c")
        s0 = wid * SB

        def idx_parts(t, sl):
            return idx_hbm.at[t * NW + wid], isr.at[sl], isem.at[sl]

        def fire_idx(t, sl):
            src, dst, sem = idx_parts(t, sl)
            pltpu.async_copy(src, dst, sem)

        def wait_idx(t, sl):
            src, dst, sem = idx_parts(t, sl)
            pltpu.make_async_copy(src, dst, sem).wait()

        def gather_parts(t, gs):
            return table_hbm.at[isr.at[gs]], gbuf.at[gs], gsem.at[gs]

        def fire_gather(t, gs):
            src, dst, sem = gather_parts(t, gs)
            pltpu.async_copy(src, dst, sem)

        def wait_gather(t, gs):
            src, dst, sem = gather_parts(t, gs)
            pltpu.make_async_copy(src, dst, sem).wait()

        def out_parts(t, os):
            return tbuf.at[os], out_hbm.at[t, :, pl.ds(s0, SB)], osem.at[os]

        def fire_out(t, os):
            src, dst, sem = out_parts(t, os)
            pltpu.async_copy(src, dst, sem)

        def wait_out(t, os):
            src, dst, sem = out_parts(t, os)
            pltpu.make_async_copy(src, dst, sem).wait()

        def transpose_block(t, gs, os):
            src = gbuf.at[gs]
            zero = jnp.zeros((16,), jnp.int32)
            for s16 in range(SB // 16):
                sl = pl.ds(s16 * 16, 16)
                rowi = jnp.arange(16, dtype=jnp.int32) + (s16 * 16)

                @plsc.parallel_loop(0, D_MODEL, unroll=4)
                def _(j):
                    tbuf[os, j, sl] = plsc.load_gather(src, [rowi, zero + j])

        def step(t, gs, os, do_gather, do_idx, do_wait_out):
            if do_gather:
                wait_idx(t + AHEAD, (gs + AHEAD) % GBUF)
                fire_gather(t + AHEAD, (gs + AHEAD) % GBUF)
            if do_idx:
                fire_idx(t + AHEAD + 1, (gs + AHEAD + 1) % GBUF)
            wait_gather(t, gs)
            if do_wait_out:
                wait_out(t - OBUF, os)
            transpose_block(t, gs, os)
            fire_out(t, os)

        for t in range(AHEAD + 1):
            fire_idx(t, t)
        for t in range(AHEAD):
            wait_idx(t, t)
            fire_gather(t, t)

        for t in range(GBUF):
            step(t, t, t % OBUF, do_gather=True, do_idx=True,
                 do_wait_out=t >= OBUF)

        def outer_body(outer, carry):
            t0 = outer * GBUF
            for k in range(GBUF):
                step(t0 + k, k, k % OBUF, do_gather=True, do_idx=True,
                     do_wait_out=True)
            return carry

        lax.fori_loop(1, n_t // GBUF - 1, outer_body, 0)

        for t in range(n_t - GBUF, n_t):
            step(t, t % GBUF, t % OBUF, do_gather=t + AHEAD < n_t,
                 do_idx=t + AHEAD + 1 < n_t, do_wait_out=True)

        for t in range(n_t - OBUF, n_t):
            wait_out(t, t % OBUF)

    return gather_k


def kernel(input_sequence, embedding_table):
    S, T_COLS = input_sequence.shape
    table_t = embedding_table.T
    idx_t = input_sequence.T.astype(jnp.int32)
    idx2 = idx_t.reshape(T_COLS * (S // 128), 128)
    table_r = _tc_repack(table_t)
    out_t = _make_sc_gather(T_COLS, S)(table_r, idx2)
    return jnp.transpose(out_t, (2, 0, 1))


# speedup vs baseline: 2.0169x; 1.6402x over previous
import functools
import jax
import jax.numpy as jnp
from jax import lax
from jax.experimental import pallas as pl
from jax.experimental.pallas import tpu as pltpu
from jax.experimental.pallas import tpu_sc as plsc

D_MODEL = 64
SCALE = 8.0
BC = 2048
N_PAD = 489 * BC


def _tc_repack(table_t):
    def repack_kernel(a_ref, out_ref):
        out_ref[:, 0:D_MODEL] = a_ref[...].T * SCALE

    return pl.pallas_call(
        repack_kernel,
        grid=(N_PAD // BC,),
        in_specs=[pl.BlockSpec((D_MODEL, BC), lambda b: (0, b))],
        out_specs=pl.BlockSpec((BC, 128), lambda b: (b, 0)),
        out_shape=jax.ShapeDtypeStruct((N_PAD, 128), jnp.float32),
    )(table_t)


def _make_sc_gather(T_COLS, S):
    NC, NS = 2, 16
    NW = NC * NS
    SB = S // NW
    n_t = T_COLS
    GBUF = 4
    AHEAD = 2
    OBUF = 2

    mesh = plsc.VectorSubcoreMesh(core_axis_name="c", subcore_axis_name="s")

    @functools.partial(
        pl.kernel,
        mesh=mesh,
        out_type=jax.ShapeDtypeStruct((T_COLS, D_MODEL, S), jnp.float32),
        scratch_types=[
            pltpu.VMEM((GBUF, SB), jnp.int32),
            pltpu.VMEM((GBUF, SB, 128), jnp.float32),
            pltpu.VMEM((OBUF, D_MODEL, SB), jnp.float32),
            pltpu.SemaphoreType.DMA((GBUF,)),
            pltpu.SemaphoreType.DMA((GBUF,)),
            pltpu.SemaphoreType.DMA((OBUF,)),
        ],
        compiler_params=pltpu.CompilerParams(
            use_tc_tiling_on_sc=True, needs_layout_passes=False
        ),
    )
    def gather_k(table_hbm, idx_hbm, out_hbm, isr, gbuf, tbuf,
                 isem, gsem, osem):
        wid = lax.axis_index("s") * NC + lax.axis_index("c")
        s0 = wid * SB

        def idx_parts(t, sl):
            return idx_hbm.at[t * NW + wid], isr.at[sl], isem.at[sl]

        def fire_idx(t, sl):
            src, dst, sem = idx_parts(t, sl)
            pltpu.async_copy(src, dst, sem)

        def wait_idx(t, sl):
            src, dst, sem = idx_parts(t, sl)
            pltpu.make_async_copy(src, dst, sem).wait()

        def gather_parts(t, gs):
            return table_hbm.at[isr.at[gs]], gbuf.at[gs], gsem.at[gs]

        def fire_gather(t, gs):
            src, dst, sem = gather_parts(t, gs)
            pltpu.async_copy(src, dst, sem)

        def wait_gather(t, gs):
            src, dst, sem = gather_parts(t, gs)
            pltpu.make_async_copy(src, dst, sem).wait()

        def out_parts(t, os):
            return tbuf.at[os], out_hbm.at[t, :, pl.ds(s0, SB)], osem.at[os]

        def fire_out(t, os):
            src, dst, sem = out_parts(t, os)
            pltpu.async_copy(src, dst, sem)

        def wait_out(t, os):
            src, dst, sem = out_parts(t, os)
            pltpu.make_async_copy(src, dst, sem).wait()

        zero = jnp.zeros((16,), jnp.int32)

        iota = jnp.arange(16, dtype=jnp.int32)
        rot = [(iota + d) & 15 for d in range(16)]

        def transpose_block(t, gs, os):
            src = gbuf.at[gs]
            dst = tbuf.at[os]

            @plsc.parallel_loop(0, SB, unroll=2)
            def _(i):
                rowi = iota + (i & ~15)
                rotd = (iota + (i & 15)) & 15
                for j16 in range(D_MODEL // 16):
                    col = rotd + (j16 * 16)
                    v = plsc.load_gather(src, [rowi, col])
                    plsc.store_scatter(dst, [col, rowi], v)

        def step(t, gs, os, do_gather, do_idx, do_wait_out):
            if do_gather:
                wait_idx(t + AHEAD, (gs + AHEAD) % GBUF)
                fire_gather(t + AHEAD, (gs + AHEAD) % GBUF)
            if do_idx:
                fire_idx(t + AHEAD + 1, (gs + AHEAD + 1) % GBUF)
            wait_gather(t, gs)
            if do_wait_out:
                wait_out(t - OBUF, os)
            transpose_block(t, gs, os)
            fire_out(t, os)

        for t in range(AHEAD + 1):
            fire_idx(t, t)
        for t in range(AHEAD):
            wait_idx(t, t)
            fire_gather(t, t)

        for t in range(GBUF):
            step(t, t, t % OBUF, do_gather=True, do_idx=True,
                 do_wait_out=t >= OBUF)

        def outer_body(outer, carry):
            t0 = outer * GBUF
            for k in range(GBUF):
                step(t0 + k, k, k % OBUF, do_gather=True, do_idx=True,
                     do_wait_out=True)
            return carry

        lax.fori_loop(1, n_t // GBUF - 1, outer_body, 0)

        for t in range(n_t - GBUF, n_t):
            step(t, t % GBUF, t % OBUF, do_gather=t + AHEAD < n_t,
                 do_idx=t + AHEAD + 1 < n_t, do_wait_out=True)

        for t in range(n_t - OBUF, n_t):
            wait_out(t, t % OBUF)

    return gather_k


def kernel(input_sequence, embedding_table):
    S, T_COLS = input_sequence.shape
    table_t = embedding_table.T
    idx_t = input_sequence.T.astype(jnp.int32)
    idx2 = idx_t.reshape(T_COLS * (S // 128), 128)
    table_r = _tc_repack(table_t)
    out_t = _make_sc_gather(T_COLS, S)(table_r, idx2)
    return jnp.transpose(out_t, (2, 0, 1))
